# baseline (speedup 1.0000x reference)
import functools

import jax
import jax.numpy as jnp
from jax import lax
from jax.experimental import pallas as pl
from jax.experimental.pallas import tpu as pltpu
from jax.experimental.pallas import tpu_sc as plsc

CA = 527
CV = 1000
SEG = 256
ROIS = 8
K_TOP = 13
KA = 5
KV = 10
CAP = 528
CVP = 1024
RU = 272


def _passmask(data, levels_ref):
    rows, cols = data.shape
    m = jnp.max(data, axis=1, keepdims=True)
    fiota = jax.lax.broadcasted_iota(
        jnp.int32, (rows, cols), 1).astype(jnp.float32)
    work = data
    for k in range(K_TOP):
        mk = jnp.max(work, axis=1, keepdims=True)
        first = jnp.min(jnp.where(work == mk, fiota, jnp.float32(cols)),
                        axis=1, keepdims=True)
        ok = (mk / m) * levels_ref[0, k] >= 0.5
        firstq = jnp.where(ok, first, jnp.float32(-1.0))
        work = jnp.where(fiota == firstq, -jnp.inf, work)
    return jnp.where(work == -jnp.inf, 1.0, 0.0)


def _colpad(x, width, fill=0.0):
    rows = x.shape[0]
    pad = jnp.full((rows, width - x.shape[1]), fill, jnp.float32)
    return jnp.concatenate([x, pad], axis=1)


def _score_body(levels_ref, lr_ref, lv_ref, out_ref):
    b = pl.program_id(0)

    @pl.when(b < 8)
    def _():
        pm = _passmask(_colpad(lr_ref[...], CVP), levels_ref)
        out_ref[0] = jnp.max(pm.reshape(32, ROIS, CVP), axis=1)

    @pl.when(b == 8)
    def _():
        pm = _passmask(_colpad(lv_ref[...], CVP), levels_ref)
        out_ref[0] = jnp.zeros((32, CVP), jnp.float32)
        out_ref[0, 0:1] = jnp.max(pm, axis=0, keepdims=True)


def _stat(mask, k):
    n = mask.shape[1]
    ri = jax.lax.broadcasted_iota(jnp.int32, (n, n), 0)
    ci = jax.lax.broadcasted_iota(jnp.int32, (n, n), 1)
    tri = (ri <= ci).astype(jnp.bfloat16)
    cum = jax.lax.dot_general(mask.astype(jnp.bfloat16), tri,
                              (((1,), (0,)), ((), ())),
                              preferred_element_type=jnp.float32)
    kept = jnp.where(cum <= k, mask, 0.0)
    return jnp.sum(kept, axis=0, keepdims=True)


def _mask_body(vseg_ref, lt_ref, la_ref, va_ref, aa_ref, sa_ref, sv_ref):
    vam = jnp.concatenate(
        [vseg_ref[0:8].reshape(SEG, CVP), vseg_ref[8, 0:1],
         jnp.zeros((RU - SEG - 1, CVP), jnp.float32)], axis=0)
    va_ref[...] = vam
    lt = _colpad(lt_ref[...], CAP, -1.0)
    thr = jnp.minimum(jnp.float32(0.4), jnp.max(lt, axis=1, keepdims=True))
    la = _colpad(la_ref[...], CAP, -1.0)
    thrf = jnp.minimum(jnp.float32(0.4), jnp.max(la))
    aam = jnp.concatenate(
        [(lt >= thr).astype(jnp.float32),
         (la[0:1] >= thrf).astype(jnp.float32),
         jnp.zeros((RU - SEG - 1, CAP), jnp.float32)], axis=0)
    aa_ref[...] = aam
    sa_ref[...] = _stat(aam, KA)
    sv_ref[...] = _stat(vam, KV)


NC = 2
NS = 16
RPT = RU // NS
CVR = 63
CAR = CAP // NS
CA_W = CAR * CAP
CV_W = CVR * CVP
BIG = 1 << 20


def _sc_accum_body(am_hbm, vm_hbm, ca_hbm, cv_hbm, cav_hbm,
                   rows_v, rows_a, buf, ivloc, ialoc, iv_all, ia_all,
                   shard, sp_iv, sp_ia):
    c = lax.axis_index("c")
    s = lax.axis_index("s")
    row_lo = s * RPT
    iota16 = lax.iota(jnp.int32, 16)
    ones16 = jnp.full((16,), 1.0, jnp.float32)
    zeros16 = jnp.zeros((16,), jnp.float32)

    def compact(rows_ref, ncols, k, outloc):
        nch = ncols // 16
        kv = jnp.full((16,), k, jnp.int32)
        bigv = jnp.full((16,), BIG, jnp.int32)
        for rl in range(RPT):
            outloc[pl.ds(rl * 16, 16)] = bigv

            def body(ch, cnt):
                v = rows_ref[pl.ds(rl * ncols + ch * 16, 16)]
                m = v > 0.5
                cum = plsc.cumsum(m.astype(jnp.int32))
                pos = cnt + cum - 1
                plsc.store_scatter(buf, [pos], iota16 + ch * 16,
                                   mask=m & (pos < 16))
                return cnt + plsc.all_reduce_population_count(m)

            @pl.when(row_lo + rl < SEG + 1)
            def _():
                def group(g, cnt):
                    def run(cnt):
                        return lax.fori_loop(g * 8, (g + 1) * 8, body, cnt)

                    return lax.cond(jnp.max(cnt) < k, run, lambda c: c, cnt)

                cnt = lax.fori_loop(0, nch // 8, group,
                                    jnp.zeros((16,), jnp.int32))
                if nch % 8:
                    cnt = lax.fori_loop(8 * (nch // 8), nch, body, cnt)
                raw = buf[...]
                valid = iota16 < jnp.minimum(cnt, kv)
                outloc[pl.ds(rl * 16, 16)] = jnp.where(valid, raw, BIG)

    with jax.named_scope("sc_compact"):
        pltpu.sync_copy(vm_hbm.at[pl.ds(row_lo * CVP, RPT * CVP)], rows_v)
        compact(rows_v, CVP, KV, ivloc)
        pltpu.sync_copy(ivloc, sp_iv.at[pl.ds(row_lo * 16, RPT * 16)])

        @pl.when(c == 1)
        def _():
            pltpu.sync_copy(am_hbm.at[pl.ds(row_lo * CAP, RPT * CAP)], rows_a)
            compact(rows_a, CAP, KA, ialoc)
            pltpu.sync_copy(ialoc, sp_ia.at[pl.ds(row_lo * 16, RPT * 16)])

    with jax.named_scope("sc_zero"):
        def zbody(i, carry):
            for t in range(8):
                shard[pl.ds(i * 128 + t * 16, 16)] = zeros16
            return carry

        lax.fori_loop(0, CV_W // 128, zbody, 0)

    plsc.subcore_barrier()
    pltpu.sync_copy(sp_iv, iv_all)

    @pl.when(c == 1)
    def _():
        pltpu.sync_copy(sp_ia, ia_all)

    @pl.when(c == 0)
    def _():
      with jax.named_scope("sc_accum_v"):
        lo = s * CVR

        def body(u, carry):
            iv = iv_all[pl.ds(u * 16, 16)]
            validv = iv < CV
            hits = plsc.all_reduce_population_count(
                validv & (iv >= lo) & (iv < lo + CVR))

            @pl.when(hits[0] > 0)
            def _():
                for j in range(KV):
                    rj = iv[j]

                    @pl.when((rj >= lo) & (rj < lo + CVR))
                    def _():
                        flat = iv + (rj - lo) * CV
                        plsc.addupdate_scatter(shard, [flat], ones16,
                                               mask=validv)
            return carry

        lax.fori_loop(0, SEG + 1, body, 0)
        with jax.named_scope("sc_copyout_v"):
            @pl.when(s < NS - 1)
            def _():
                pltpu.sync_copy(shard.at[pl.ds(0, CVR * CV)],
                                cv_hbm.at[pl.ds(lo * CV, CVR * CV)])

            @pl.when(s == NS - 1)
            def _():
                pltpu.sync_copy(shard.at[pl.ds(0, (CV - 15 * CVR) * CV)],
                                cv_hbm.at[pl.ds(15 * CVR * CV,
                                                (CV - 15 * CVR) * CV)])

    @pl.when(c == 1)
    def _():
      with jax.named_scope("sc_accum_a"):
        lo = s * CAR

        def body(u, carry):
            ia = ia_all[pl.ds(u * 16, 16)]
            valida = ia < CA
            hits = plsc.all_reduce_population_count(
                valida & (ia >= lo) & (ia < lo + CAR))

            @pl.when(hits[0] > 0)
            def _():
                iv = iv_all[pl.ds(u * 16, 16)]
                validv = iv < CV
                for j in range(KA):
                    rj = ia[j]

                    @pl.when((rj >= lo) & (rj < lo + CAR))
                    def _():
                        flata = ia + (rj - lo) * CAP
                        flatv = iv + (rj - lo) * CV + CA_W
                        plsc.addupdate_scatter(shard, [flata], ones16,
                                               mask=valida)
                        plsc.addupdate_scatter(shard, [flatv], ones16,
                                               mask=validv)
            return carry

        lax.fori_loop(0, SEG + 1, body, 0)
        pltpu.sync_copy(shard.at[pl.ds(0, CA_W)],
                        ca_hbm.at[pl.ds(lo * CAP, CAR * CAP)])

        @pl.when(s < NS - 1)
        def _():
            pltpu.sync_copy(shard.at[pl.ds(CA_W, CAR * CV)],
                            cav_hbm.at[pl.ds(lo * CV, CAR * CV)])

        @pl.when(s == NS - 1)
        def _():
            pltpu.sync_copy(shard.at[pl.ds(CA_W, (CA - 15 * CAR) * CV)],
                            cav_hbm.at[pl.ds(15 * CAR * CV,
                                             (CA - 15 * CAR) * CV)])


@jax.jit
def kernel(label_a, label_t, label_v, label_r):
    levels = jnp.power(jnp.float32(0.95),
                       jnp.arange(1.0, 17.0, dtype=jnp.float32))[None, :]

    vseg = pl.pallas_call(
        _score_body,
        grid=(9,),
        in_specs=[
            pl.BlockSpec((1, 16), lambda b: (0, 0)),
            pl.BlockSpec((SEG, CV), lambda b: (jnp.minimum(b, 7), 0)),
            pl.BlockSpec((SEG, CV), lambda b: (0, 0)),
        ],
        out_specs=pl.BlockSpec((1, 32, CVP), lambda b: (b, 0, 0)),
        out_shape=jax.ShapeDtypeStruct((9, 32, CVP), jnp.float32),
    )(levels, label_r, label_v)

    vmask, amask, sa, sv = pl.pallas_call(
        _mask_body,
        out_shape=(
            jax.ShapeDtypeStruct((RU, CVP), jnp.float32),
            jax.ShapeDtypeStruct((RU, CAP), jnp.float32),
            jax.ShapeDtypeStruct((1, CAP), jnp.float32),
            jax.ShapeDtypeStruct((1, CVP), jnp.float32),
        ),
    )(vseg, label_t, label_a)

    accum = pl.kernel(
        _sc_accum_body,
        out_type=(
            jax.ShapeDtypeStruct((CAP * CAP,), jnp.float32),
            jax.ShapeDtypeStruct((CV * CV,), jnp.float32),
            jax.ShapeDtypeStruct((CA * CV,), jnp.float32),
        ),
        mesh=plsc.VectorSubcoreMesh(core_axis_name="c", subcore_axis_name="s",
                                    num_cores=NC, num_subcores=NS),
        compiler_params=pltpu.CompilerParams(needs_layout_passes=False),
        scratch_types=[
            pltpu.VMEM((RPT * CVP,), jnp.float32),
            pltpu.VMEM((RPT * CAP,), jnp.float32),
            pltpu.VMEM((16,), jnp.int32),
            pltpu.VMEM((RPT * 16,), jnp.int32),
            pltpu.VMEM((RPT * 16,), jnp.int32),
            pltpu.VMEM((RU * 16,), jnp.int32),
            pltpu.VMEM((RU * 16,), jnp.int32),
            pltpu.VMEM((CV_W,), jnp.float32),
            pltpu.VMEM_SHARED((RU * 16,), jnp.int32),
            pltpu.VMEM_SHARED((RU * 16,), jnp.int32),
        ],
    )
    ca, cv, cav = accum(amask.reshape(-1), vmask.reshape(-1))

    return (sa[0, :CA], sv[0, :CV],
            ca.reshape(CAP, CAP)[:CA, :CA],
            cv.reshape(CV, CV),
            cav.reshape(CA, CV))


# speedup vs baseline: 1.0099x; 1.0099x over previous
import functools

import jax
import jax.numpy as jnp
from jax import lax
from jax.experimental import pallas as pl
from jax.experimental.pallas import tpu as pltpu
from jax.experimental.pallas import tpu_sc as plsc

CA = 527
CV = 1000
SEG = 256
ROIS = 8
K_TOP = 13
KA = 5
KV = 10
CAP = 528
CVP = 1024
RU = 272


def _passmask(data, levels_ref):
    rows, cols = data.shape
    m = jnp.max(data, axis=1, keepdims=True)
    fiota = jax.lax.broadcasted_iota(
        jnp.int32, (rows, cols), 1).astype(jnp.float32)
    work = data
    for k in range(K_TOP):
        mk = jnp.max(work, axis=1, keepdims=True)
        first = jnp.min(jnp.where(work == mk, fiota, jnp.float32(cols)),
                        axis=1, keepdims=True)
        ok = (mk / m) * levels_ref[0, k] >= 0.5
        firstq = jnp.where(ok, first, jnp.float32(-1.0))
        work = jnp.where(fiota == firstq, -jnp.inf, work)
    return jnp.where(work == -jnp.inf, 1.0, 0.0)


def _colpad(x, width, fill=0.0):
    rows = x.shape[0]
    pad = jnp.full((rows, width - x.shape[1]), fill, jnp.float32)
    return jnp.concatenate([x, pad], axis=1)


def _score_body(levels_ref, lr_ref, lv_ref, out_ref):
    b = pl.program_id(0)

    @pl.when(b < 8)
    def _():
        pm = _passmask(_colpad(lr_ref[...], CVP), levels_ref)
        out_ref[0] = jnp.max(pm.reshape(32, ROIS, CVP), axis=1)

    @pl.when(b == 8)
    def _():
        pm = _passmask(_colpad(lv_ref[...], CVP), levels_ref)
        out_ref[0] = jnp.zeros((32, CVP), jnp.float32)
        out_ref[0, 0:1] = jnp.max(pm, axis=0, keepdims=True)


def _stat(mask, k):
    n = mask.shape[1]
    ri = jax.lax.broadcasted_iota(jnp.int32, (n, n), 0)
    ci = jax.lax.broadcasted_iota(jnp.int32, (n, n), 1)
    tri = (ri <= ci).astype(jnp.bfloat16)
    cum = jax.lax.dot_general(mask.astype(jnp.bfloat16), tri,
                              (((1,), (0,)), ((), ())),
                              preferred_element_type=jnp.float32)
    kept = jnp.where(cum <= k, mask, 0.0)
    return jnp.sum(kept, axis=0, keepdims=True)


def _mask_body(vseg_ref, lt_ref, la_ref, va_ref, aa_ref, sa_ref, sv_ref):
    vam = jnp.concatenate(
        [vseg_ref[0:8].reshape(SEG, CVP), vseg_ref[8, 0:1],
         jnp.zeros((RU - SEG - 1, CVP), jnp.float32)], axis=0)
    va_ref[...] = vam
    lt = _colpad(lt_ref[...], CAP, -1.0)
    thr = jnp.minimum(jnp.float32(0.4), jnp.max(lt, axis=1, keepdims=True))
    la = _colpad(la_ref[...], CAP, -1.0)
    thrf = jnp.minimum(jnp.float32(0.4), jnp.max(la))
    aam = jnp.concatenate(
        [(lt >= thr).astype(jnp.float32),
         (la[0:1] >= thrf).astype(jnp.float32),
         jnp.zeros((RU - SEG - 1, CAP), jnp.float32)], axis=0)
    aa_ref[...] = aam
    sa_ref[...] = _stat(aam, KA)
    sv_ref[...] = _stat(vam, KV)


NC = 2
NS = 16
RPT = RU // NS
CVR = 63
CAR = CAP // NS
CA_W = CAR * CAP
CV_W = CVR * CVP
BIG = 1 << 20


def _sc_accum_body(am_hbm, vm_hbm, ca_hbm, cv_hbm, cav_hbm,
                   rows_v, rows_a, buf, ivloc, ialoc, iv_all, ia_all,
                   shard, sp_iv, sp_ia, sem):
    c = lax.axis_index("c")
    s = lax.axis_index("s")
    row_lo = s * RPT
    iota16 = lax.iota(jnp.int32, 16)
    ones16 = jnp.full((16,), 1.0, jnp.float32)
    zeros16 = jnp.zeros((16,), jnp.float32)

    def compact(rows_ref, ncols, k, outloc):
        nch = ncols // 16
        kv = jnp.full((16,), k, jnp.int32)
        bigv = jnp.full((16,), BIG, jnp.int32)
        for rl in range(RPT):
            outloc[pl.ds(rl * 16, 16)] = bigv

            def body(ch, cnt):
                v = rows_ref[pl.ds(rl * ncols + ch * 16, 16)]
                m = v > 0.5
                cum = plsc.cumsum(m.astype(jnp.int32))
                pos = cnt + cum - 1
                plsc.store_scatter(buf, [pos], iota16 + ch * 16,
                                   mask=m & (pos < 16))
                return cnt + plsc.all_reduce_population_count(m)

            @pl.when(row_lo + rl < SEG + 1)
            def _():
                def group(g, cnt):
                    def run(cnt):
                        return lax.fori_loop(g * 8, (g + 1) * 8, body, cnt)

                    return lax.cond(jnp.max(cnt) < k, run, lambda c: c, cnt)

                cnt = lax.fori_loop(0, nch // 8, group,
                                    jnp.zeros((16,), jnp.int32))
                if nch % 8:
                    cnt = lax.fori_loop(8 * (nch // 8), nch, body, cnt)
                raw = buf[...]
                valid = iota16 < jnp.minimum(cnt, kv)
                outloc[pl.ds(rl * 16, 16)] = jnp.where(valid, raw, BIG)

    with jax.named_scope("sc_compact"):
        pltpu.sync_copy(vm_hbm.at[pl.ds(row_lo * CVP, RPT * CVP)], rows_v)
        compact(rows_v, CVP, KV, ivloc)
        pltpu.sync_copy(ivloc, sp_iv.at[pl.ds(row_lo * 16, RPT * 16)])

        @pl.when(c == 1)
        def _():
            pltpu.sync_copy(am_hbm.at[pl.ds(row_lo * CAP, RPT * CAP)], rows_a)
            compact(rows_a, CAP, KA, ialoc)
            pltpu.sync_copy(ialoc, sp_ia.at[pl.ds(row_lo * 16, RPT * 16)])

    with jax.named_scope("sc_zero"):
        def zbody(i, carry):
            for t in range(8):
                shard[pl.ds(i * 128 + t * 16, 16)] = zeros16
            return carry

        lax.fori_loop(0, CV_W // 128, zbody, 0)

    plsc.subcore_barrier()
    pltpu.sync_copy(sp_iv, iv_all)

    @pl.when(c == 1)
    def _():
        pltpu.sync_copy(sp_ia, ia_all)

    @pl.when(c == 0)
    def _():
      with jax.named_scope("sc_accum_v"):


        def body(u, carry):
            iv = iv_all[pl.ds(u * 16, 16)]
            validv = iv < CV
            hits = plsc.all_reduce_population_count(
                validv & ((iv & 15) == s))

            @pl.when(hits[0] > 0)
            def _():
                for j in range(KV):
                    rj = iv[j]

                    @pl.when(((rj & 15) == s) & (rj < CV))
                    def _():
                        flat = iv + (rj >> 4) * CV
                        plsc.addupdate_scatter(shard, [flat], ones16,
                                               mask=validv)
            return carry

        lax.fori_loop(0, SEG + 1, body, 0)
        with jax.named_scope("sc_copyout_v"):
            handles = []
            for t in range(62):
                handles.append(pltpu.async_copy(
                    shard.at[pl.ds(t * CV, CV)],
                    cv_hbm.at[pl.ds((t * 16 + s) * CV, CV)], sem))
            for h in handles:
                h.wait()

            @pl.when(s < 8)
            def _():
                pltpu.async_copy(
                    shard.at[pl.ds(62 * CV, CV)],
                    cv_hbm.at[pl.ds((62 * 16 + s) * CV, CV)], sem).wait()

    @pl.when(c == 1)
    def _():
      with jax.named_scope("sc_accum_a"):


        def body(u, carry):
            ia = ia_all[pl.ds(u * 16, 16)]
            valida = ia < CA
            hits = plsc.all_reduce_population_count(
                valida & ((ia & 15) == s))

            @pl.when(hits[0] > 0)
            def _():
                iv = iv_all[pl.ds(u * 16, 16)]
                validv = iv < CV
                for j in range(KA):
                    rj = ia[j]

                    @pl.when(((rj & 15) == s) & (rj < CA))
                    def _():
                        flata = ia + (rj >> 4) * CAP
                        flatv = iv + (rj >> 4) * CV + CA_W
                        plsc.addupdate_scatter(shard, [flata], ones16,
                                               mask=valida)
                        plsc.addupdate_scatter(shard, [flatv], ones16,
                                               mask=validv)
            return carry

        lax.fori_loop(0, SEG + 1, body, 0)
        handles = []
        for t in range(32):
            handles.append(pltpu.async_copy(
                shard.at[pl.ds(t * CAP, CAP)],
                ca_hbm.at[pl.ds((t * 16 + s) * CAP, CAP)], sem))
            handles.append(pltpu.async_copy(
                shard.at[pl.ds(CA_W + t * CV, CV)],
                cav_hbm.at[pl.ds((t * 16 + s) * CV, CV)], sem))
        for h in handles:
            h.wait()

        @pl.when(s < 15)
        def _():
            pltpu.async_copy(
                shard.at[pl.ds(32 * CAP, CAP)],
                ca_hbm.at[pl.ds((32 * 16 + s) * CAP, CAP)], sem).wait()
            pltpu.async_copy(
                shard.at[pl.ds(CA_W + 32 * CV, CV)],
                cav_hbm.at[pl.ds((32 * 16 + s) * CV, CV)], sem).wait()


@jax.jit
def kernel(label_a, label_t, label_v, label_r):
    levels = jnp.power(jnp.float32(0.95),
                       jnp.arange(1.0, 17.0, dtype=jnp.float32))[None, :]

    vseg = pl.pallas_call(
        _score_body,
        grid=(9,),
        in_specs=[
            pl.BlockSpec((1, 16), lambda b: (0, 0)),
            pl.BlockSpec((SEG, CV), lambda b: (jnp.minimum(b, 7), 0)),
            pl.BlockSpec((SEG, CV), lambda b: (0, 0)),
        ],
        out_specs=pl.BlockSpec((1, 32, CVP), lambda b: (b, 0, 0)),
        out_shape=jax.ShapeDtypeStruct((9, 32, CVP), jnp.float32),
    )(levels, label_r, label_v)

    vmask, amask, sa, sv = pl.pallas_call(
        _mask_body,
        out_shape=(
            jax.ShapeDtypeStruct((RU, CVP), jnp.float32),
            jax.ShapeDtypeStruct((RU, CAP), jnp.float32),
            jax.ShapeDtypeStruct((1, CAP), jnp.float32),
            jax.ShapeDtypeStruct((1, CVP), jnp.float32),
        ),
    )(vseg, label_t, label_a)

    accum = pl.kernel(
        _sc_accum_body,
        out_type=(
            jax.ShapeDtypeStruct((CAP * CAP,), jnp.float32),
            jax.ShapeDtypeStruct((CV * CV,), jnp.float32),
            jax.ShapeDtypeStruct((CA * CV,), jnp.float32),
        ),
        mesh=plsc.VectorSubcoreMesh(core_axis_name="c", subcore_axis_name="s",
                                    num_cores=NC, num_subcores=NS),
        compiler_params=pltpu.CompilerParams(needs_layout_passes=False),
        scratch_types=[
            pltpu.VMEM((RPT * CVP,), jnp.float32),
            pltpu.VMEM((RPT * CAP,), jnp.float32),
            pltpu.VMEM((16,), jnp.int32),
            pltpu.VMEM((RPT * 16,), jnp.int32),
            pltpu.VMEM((RPT * 16,), jnp.int32),
            pltpu.VMEM((RU * 16,), jnp.int32),
            pltpu.VMEM((RU * 16,), jnp.int32),
            pltpu.VMEM((CV_W,), jnp.float32),
            pltpu.VMEM_SHARED((RU * 16,), jnp.int32),
            pltpu.VMEM_SHARED((RU * 16,), jnp.int32),
            pltpu.SemaphoreType.DMA,
        ],
    )
    ca, cv, cav = accum(amask.reshape(-1), vmask.reshape(-1))

    return (sa[0, :CA], sv[0, :CV],
            ca.reshape(CAP, CAP)[:CA, :CA],
            cv.reshape(CV, CV),
            cav.reshape(CA, CV))


# speedup vs baseline: 1.0103x; 1.0004x over previous
import jax
import jax.numpy as jnp
from jax import lax
from jax.experimental import pallas as pl
from jax.experimental.pallas import tpu as pltpu
from jax.experimental.pallas import tpu_sc as plsc

CA = 527
CV = 1000
SEG = 256
ROIS = 8
K_TOP = 13
KA = 5
KV = 10
CAP = 528
CVP = 1024
RU = 272


def _passmask(data, levels_ref):
    rows, cols = data.shape
    m = jnp.max(data, axis=1, keepdims=True)
    fiota = jax.lax.broadcasted_iota(
        jnp.int32, (rows, cols), 1).astype(jnp.float32)
    work = data
    for k in range(K_TOP):
        mk = jnp.max(work, axis=1, keepdims=True)
        first = jnp.min(jnp.where(work == mk, fiota, jnp.float32(cols)),
                        axis=1, keepdims=True)
        ok = (mk / m) * levels_ref[0, k] >= 0.5
        firstq = jnp.where(ok, first, jnp.float32(-1.0))
        work = jnp.where(fiota == firstq, -jnp.inf, work)
    return jnp.where(work == -jnp.inf, 1.0, 0.0)


def _colpad(x, width, fill=0.0):
    rows = x.shape[0]
    pad = jnp.full((rows, width - x.shape[1]), fill, jnp.float32)
    return jnp.concatenate([x, pad], axis=1)


def _score_body(levels_ref, lr_ref, lv_ref, out_ref):
    b = pl.program_id(0)

    @pl.when(b < 8)
    def _():
        pm = _passmask(_colpad(lr_ref[...], CVP), levels_ref)
        out_ref[0] = jnp.max(pm.reshape(32, ROIS, CVP), axis=1)

    @pl.when(b == 8)
    def _():
        pm = _passmask(_colpad(lv_ref[...], CVP), levels_ref)
        out_ref[0] = jnp.zeros((32, CVP), jnp.float32)
        out_ref[0, 0:1] = jnp.max(pm, axis=0, keepdims=True)


def _stat(mask, k):
    n = mask.shape[1]
    ri = jax.lax.broadcasted_iota(jnp.int32, (n, n), 0)
    ci = jax.lax.broadcasted_iota(jnp.int32, (n, n), 1)
    tri = (ri <= ci).astype(jnp.bfloat16)
    cum = jax.lax.dot_general(mask.astype(jnp.bfloat16), tri,
                              (((1,), (0,)), ((), ())),
                              preferred_element_type=jnp.float32)
    kept = jnp.where(cum <= k, mask, 0.0)
    return jnp.sum(kept, axis=0, keepdims=True)


def _mask_body(vseg_ref, lt_ref, la_ref, va_ref, aa_ref, sa_ref, sv_ref):
    vam = jnp.concatenate(
        [vseg_ref[0:8].reshape(SEG, CVP), vseg_ref[8, 0:1],
         jnp.zeros((RU - SEG - 1, CVP), jnp.float32)], axis=0)
    va_ref[...] = vam
    lt = _colpad(lt_ref[...], CAP, -1.0)
    thr = jnp.minimum(jnp.float32(0.4), jnp.max(lt, axis=1, keepdims=True))
    la = _colpad(la_ref[...], CAP, -1.0)
    thrf = jnp.minimum(jnp.float32(0.4), jnp.max(la))
    aam = jnp.concatenate(
        [(lt >= thr).astype(jnp.float32),
         (la[0:1] >= thrf).astype(jnp.float32),
         jnp.zeros((RU - SEG - 1, CAP), jnp.float32)], axis=0)
    aa_ref[...] = aam
    sa_ref[...] = _stat(aam, KA)
    sv_ref[...] = _stat(vam, KV)


NC = 2
NS = 16
RPT = RU // NS
CVR = 63
CAR = CAP // NS
CA_W = CAR * CAP
CV_W = CVR * CVP
BIG = 1 << 20


def _sc_accum_body(am_hbm, vm_hbm, ca_hbm, cv_hbm, cav_hbm,
                   rows_v, rows_a, buf, ivloc, ialoc, iv_all, ia_all,
                   shard, sp_iv, sp_ia, sem):
    c = lax.axis_index("c")
    s = lax.axis_index("s")
    row_lo = s * RPT
    iota16 = lax.iota(jnp.int32, 16)
    ones16 = jnp.full((16,), 1.0, jnp.float32)
    zeros16 = jnp.zeros((16,), jnp.float32)

    def compact(rows_ref, ncols, k, outloc):
        nch = ncols // 16
        kv = jnp.full((16,), k, jnp.int32)
        bigv = jnp.full((16,), BIG, jnp.int32)
        for rl in range(RPT):
            outloc[pl.ds(rl * 16, 16)] = bigv

            def body(ch, cnt):
                v = rows_ref[pl.ds(rl * ncols + ch * 16, 16)]
                m = v > 0.5
                cum = plsc.cumsum(m.astype(jnp.int32))
                pos = cnt + cum - 1
                plsc.store_scatter(buf, [pos], iota16 + ch * 16,
                                   mask=m & (pos < 16))
                return cnt + plsc.all_reduce_population_count(m)

            @pl.when(row_lo + rl < SEG + 1)
            def _():
                def group(g, cnt):
                    def run(cnt):
                        return lax.fori_loop(g * 8, (g + 1) * 8, body, cnt)

                    return lax.cond(jnp.max(cnt) < k, run, lambda c: c, cnt)

                cnt = lax.fori_loop(0, nch // 8, group,
                                    jnp.zeros((16,), jnp.int32))
                if nch % 8:
                    cnt = lax.fori_loop(8 * (nch // 8), nch, body, cnt)
                raw = buf[...]
                valid = iota16 < jnp.minimum(cnt, kv)
                outloc[pl.ds(rl * 16, 16)] = jnp.where(valid, raw, BIG)

    with jax.named_scope("sc_compact"):
        pltpu.sync_copy(vm_hbm.at[pl.ds(row_lo * CVP, RPT * CVP)], rows_v)
        compact(rows_v, CVP, KV, ivloc)
        pltpu.sync_copy(ivloc, sp_iv.at[pl.ds(row_lo * 16, RPT * 16)])

        @pl.when(c == 1)
        def _():
            pltpu.sync_copy(am_hbm.at[pl.ds(row_lo * CAP, RPT * CAP)], rows_a)
            compact(rows_a, CAP, KA, ialoc)
            pltpu.sync_copy(ialoc, sp_ia.at[pl.ds(row_lo * 16, RPT * 16)])

    with jax.named_scope("sc_zero"):
        def zbody(i, carry):
            for t in range(8):
                shard[pl.ds(i * 128 + t * 16, 16)] = zeros16
            return carry

        lax.fori_loop(0, CV_W // 128, zbody, 0)

    plsc.subcore_barrier()
    pltpu.sync_copy(sp_iv, iv_all)

    @pl.when(c == 1)
    def _():
        pltpu.sync_copy(sp_ia, ia_all)

    @pl.when(c == 0)
    def _():
      with jax.named_scope("sc_accum_v"):


        def body(u, carry):
            iv = iv_all[pl.ds(u * 16, 16)]
            validv = iv < CV
            hits = plsc.all_reduce_population_count(
                validv & ((iv & 15) == s))

            @pl.when(hits[0] > 0)
            def _():
                for j in range(KV):
                    rj = iv[j]

                    @pl.when(((rj & 15) == s) & (rj < CV))
                    def _():
                        flat = iv + (rj >> 4) * CV
                        plsc.addupdate_scatter(shard, [flat], ones16,
                                               mask=validv)
            return carry

        lax.fori_loop(0, SEG + 1, body, 0)
        with jax.named_scope("sc_copyout_v"):
            handles = []
            for t in range(62):
                handles.append(pltpu.async_copy(
                    shard.at[pl.ds(t * CV, CV)],
                    cv_hbm.at[pl.ds((t * 16 + s) * CV, CV)], sem))
            for h in handles:
                h.wait()

            @pl.when(s < 8)
            def _():
                pltpu.async_copy(
                    shard.at[pl.ds(62 * CV, CV)],
                    cv_hbm.at[pl.ds((62 * 16 + s) * CV, CV)], sem).wait()

    @pl.when(c == 1)
    def _():
      with jax.named_scope("sc_accum_a"):


        def body(u, carry):
            ia = ia_all[pl.ds(u * 16, 16)]
            valida = ia < CA
            hits = plsc.all_reduce_population_count(
                valida & ((ia & 15) == s))

            @pl.when(hits[0] > 0)
            def _():
                iv = iv_all[pl.ds(u * 16, 16)]
                validv = iv < CV
                for j in range(KA):
                    rj = ia[j]

                    @pl.when(((rj & 15) == s) & (rj < CA))
                    def _():
                        flata = ia + (rj >> 4) * CAP
                        flatv = iv + (rj >> 4) * CV + CA_W
                        plsc.addupdate_scatter(shard, [flata], ones16,
                                               mask=valida)
                        plsc.addupdate_scatter(shard, [flatv], ones16,
                                               mask=validv)
            return carry

        lax.fori_loop(0, SEG + 1, body, 0)
        handles = []
        for t in range(32):
            handles.append(pltpu.async_copy(
                shard.at[pl.ds(t * CAP, CAP)],
                ca_hbm.at[pl.ds((t * 16 + s) * CAP, CAP)], sem))
            handles.append(pltpu.async_copy(
                shard.at[pl.ds(CA_W + t * CV, CV)],
                cav_hbm.at[pl.ds((t * 16 + s) * CV, CV)], sem))
        for h in handles:
            h.wait()

        @pl.when(s < 15)
        def _():
            pltpu.async_copy(
                shard.at[pl.ds(32 * CAP, CAP)],
                ca_hbm.at[pl.ds((32 * 16 + s) * CAP, CAP)], sem).wait()
            pltpu.async_copy(
                shard.at[pl.ds(CA_W + 32 * CV, CV)],
                cav_hbm.at[pl.ds((32 * 16 + s) * CV, CV)], sem).wait()


@jax.jit
def kernel(label_a, label_t, label_v, label_r):
    levels = jnp.power(jnp.float32(0.95),
                       jnp.arange(1.0, 17.0, dtype=jnp.float32))[None, :]

    vseg = pl.pallas_call(
        _score_body,
        grid=(9,),
        in_specs=[
            pl.BlockSpec((1, 16), lambda b: (0, 0)),
            pl.BlockSpec((SEG, CV), lambda b: (jnp.minimum(b, 7), 0)),
            pl.BlockSpec((SEG, CV), lambda b: (0, 0)),
        ],
        out_specs=pl.BlockSpec((1, 32, CVP), lambda b: (b, 0, 0)),
        out_shape=jax.ShapeDtypeStruct((9, 32, CVP), jnp.float32),
    )(levels, label_r, label_v)

    vmask, amask, sa, sv = pl.pallas_call(
        _mask_body,
        out_shape=(
            jax.ShapeDtypeStruct((RU, CVP), jnp.float32),
            jax.ShapeDtypeStruct((RU, CAP), jnp.float32),
            jax.ShapeDtypeStruct((1, CAP), jnp.float32),
            jax.ShapeDtypeStruct((1, CVP), jnp.float32),
        ),
    )(vseg, label_t, label_a)

    accum = pl.kernel(
        _sc_accum_body,
        out_type=(
            jax.ShapeDtypeStruct((CAP * CAP,), jnp.float32),
            jax.ShapeDtypeStruct((CV * CV,), jnp.float32),
            jax.ShapeDtypeStruct((CA * CV,), jnp.float32),
        ),
        mesh=plsc.VectorSubcoreMesh(core_axis_name="c", subcore_axis_name="s",
                                    num_cores=NC, num_subcores=NS),
        compiler_params=pltpu.CompilerParams(needs_layout_passes=False),
        scratch_types=[
            pltpu.VMEM((RPT * CVP,), jnp.float32),
            pltpu.VMEM((RPT * CAP,), jnp.float32),
            pltpu.VMEM((16,), jnp.int32),
            pltpu.VMEM((RPT * 16,), jnp.int32),
            pltpu.VMEM((RPT * 16,), jnp.int32),
            pltpu.VMEM((RU * 16,), jnp.int32),
            pltpu.VMEM((RU * 16,), jnp.int32),
            pltpu.VMEM((CV_W,), jnp.float32),
            pltpu.VMEM_SHARED((RU * 16,), jnp.int32),
            pltpu.VMEM_SHARED((RU * 16,), jnp.int32),
            pltpu.SemaphoreType.DMA,
        ],
    )
    ca, cv, cav = accum(amask.reshape(-1), vmask.reshape(-1))

    return (sa[0, :CA], sv[0, :CV],
            ca.reshape(CAP, CAP)[:CA, :CA],
            cv.reshape(CV, CV),
            cav.reshape(CA, CV))
